# baseline (speedup 1.0000x reference)
import functools

import jax
import jax.numpy as jnp
from jax import lax
from jax.experimental import pallas as pl
from jax.experimental.pallas import tpu as pltpu
from jax.experimental.pallas import tpu_sc as plsc

_B, _D, _K, _A, _H = 4096, 256, 1024, 32, 64
_BLK = 1024
_NBLK = _B // _BLK

_NC, _NS = 2, 16
_NW = _NC * _NS
_BPW = _B // _NW
_HP = 128
_IDXW = 128
_GC = 2
_GS = _BPW // _GC


def _dist_body(x_ref, cb_ref, W1_ref, b1_ref,
               idx_ref, loss_ref, cbw1_ref, cbn_scr, acc_scr):
    i = pl.program_id(0)
    cb = cb_ref[...]

    @pl.when(i == 0)
    def _():
        cbn_scr[...] = jnp.sum(cb * cb, axis=1).reshape(1, _K)
        acc_scr[...] = jnp.zeros((1, 1), jnp.float32)
        cbw1_ref[:, :_H] = (jnp.dot(cb, W1_ref[...],
                                    preferred_element_type=jnp.float32)
                            + b1_ref[...])
        cbw1_ref[:, _H:] = jnp.zeros((_K, _HP - _H), jnp.float32)

    x = x_ref[...]
    xdotc = lax.dot_general(x, cb, (((1,), (1,)), ((), ())),
                            preferred_element_type=jnp.float32)
    xn = jnp.sum(x * x, axis=1, keepdims=True)
    d = xn + cbn_scr[...] - 2.0 * xdotc
    minval = jnp.min(d, axis=1, keepdims=True)
    ids = lax.broadcasted_iota(jnp.int32, d.shape, 1)
    idx = jnp.min(jnp.where(d == minval, ids, _K), axis=1)
    idx_ref[...] = idx.reshape(_BLK // _IDXW, _IDXW)
    acc_scr[...] += jnp.sum(minval, axis=0, keepdims=True)

    @pl.when(i == _NBLK - 1)
    def _():
        loss_ref[...] = acc_scr[...] * (1.0 / (_B * _D))


_dist = pl.pallas_call(
    _dist_body,
    grid=(_NBLK,),
    in_specs=[
        pl.BlockSpec((_BLK, _D), lambda i: (i, 0)),
        pl.BlockSpec((_K, _D), lambda i: (0, 0)),
        pl.BlockSpec((_D, _H), lambda i: (0, 0)),
        pl.BlockSpec((1, _H), lambda i: (0, 0)),
    ],
    out_specs=[
        pl.BlockSpec((_BLK // _IDXW, _IDXW), lambda i: (i, 0)),
        pl.BlockSpec((1, 1), lambda i: (0, 0)),
        pl.BlockSpec((_K, _HP), lambda i: (0, 0)),
    ],
    out_shape=[
        jax.ShapeDtypeStruct((_B // _IDXW, _IDXW), jnp.int32),
        jax.ShapeDtypeStruct((1, 1), jnp.float32),
        jax.ShapeDtypeStruct((_K, _HP), jnp.float32),
    ],
    scratch_shapes=[pltpu.VMEM((1, _K), jnp.float32),
                    pltpu.VMEM((1, 1), jnp.float32)],
    compiler_params=pltpu.CompilerParams(skip_device_barrier=True),
)


@functools.cache
def _make_sc_gather():
    @functools.partial(
        pl.kernel,
        mesh=plsc.VectorSubcoreMesh(core_axis_name="c", subcore_axis_name="s",
                                    num_cores=_NC, num_subcores=_NS),
        out_type=jax.ShapeDtypeStruct((_B, _HP), jnp.float32),
        scratch_types=[
            pltpu.VMEM((_GC, _GS), jnp.int32),
            pltpu.VMEM((_BPW, _HP), jnp.float32),
            [pltpu.SemaphoreType.DMA] * _GC,
            [pltpu.SemaphoreType.DMA] * _GC,
            [pltpu.SemaphoreType.DMA] * _GC,
        ],
    )
    def _sc_gather(table_hbm, idx_hbm, out_hbm, idx_v, rows_v,
                   isems, gsems, wsems):
        wid = lax.axis_index("s") * _NC + lax.axis_index("c")
        base = wid * _BPW
        iloads = [pltpu.async_copy(
            idx_hbm.at[pl.ds(base + c * _GS, _GS)], idx_v.at[c], isems[c])
            for c in range(_GC)]
        gathers = []
        for c in range(_GC):
            iloads[c].wait()
            gathers.append(pltpu.async_copy(
                table_hbm.at[idx_v.at[c]],
                rows_v.at[pl.ds(c * _GS, _GS)], gsems[c]))
        wbs = []
        for c in range(_GC):
            gathers[c].wait()
            wbs.append(pltpu.async_copy(
                rows_v.at[pl.ds(c * _GS, _GS)],
                out_hbm.at[pl.ds(base + c * _GS, _GS)], wsems[c]))
        for c in range(_GC):
            wbs[c].wait()

    return _sc_gather


def _mlp_body(g_ref, W2_ref, b2_ref, W3_ref, b3_ref, out_ref):
    h1 = jnp.tanh(g_ref[:, :_H])
    h2 = jnp.tanh(jnp.dot(h1, W2_ref[...],
                          preferred_element_type=jnp.float32) + b2_ref[...])
    out_ref[...] = (jnp.dot(h2, W3_ref[...],
                            preferred_element_type=jnp.float32) + b3_ref[...])


_mlp = pl.pallas_call(
    _mlp_body,
    out_shape=jax.ShapeDtypeStruct((_B, _A), jnp.float32),
    compiler_params=pltpu.CompilerParams(skip_device_barrier=True),
)


@jax.jit
def kernel(x, codebook, W1, b1, W2, b2, W3, b3):
    idx2d, loss, cbw1b = _dist(x, codebook, W1, b1.reshape(1, _H))
    g = _make_sc_gather()(cbw1b, idx2d.reshape(_B))
    dist = _mlp(g, W2, b2.reshape(1, _H), W3, b3.reshape(1, _A))
    return dist, loss.reshape(())


# speedup vs baseline: 1.0147x; 1.0147x over previous
import functools

import jax
import jax.numpy as jnp
from jax import lax
from jax.experimental import pallas as pl
from jax.experimental.pallas import tpu as pltpu
from jax.experimental.pallas import tpu_sc as plsc

_B, _D, _K, _A, _H = 4096, 256, 1024, 32, 64
_BLK = 2048
_NBLK = _B // _BLK

_NC, _NS = 2, 16
_NW = _NC * _NS
_BPW = _B // _NW
_HP = 128
_IDXW = 128
_GC = 2
_GS = _BPW // _GC


def _dist_body(x_ref, cb_ref, W1_ref, b1_ref,
               idx_ref, loss_ref, cbw1_ref, cbn_scr, acc_scr):
    i = pl.program_id(0)
    cb = cb_ref[...]

    @pl.when(i == 0)
    def _():
        cbn_scr[...] = jnp.sum(cb * cb, axis=1).reshape(1, _K)
        acc_scr[...] = jnp.zeros((1, 1), jnp.float32)
        cbw1_ref[:, :_H] = (jnp.dot(cb, W1_ref[...],
                                    preferred_element_type=jnp.float32)
                            + b1_ref[...])
        cbw1_ref[:, _H:] = jnp.zeros((_K, _HP - _H), jnp.float32)

    x = x_ref[...]
    xdotc = lax.dot_general(x, cb, (((1,), (1,)), ((), ())),
                            preferred_element_type=jnp.float32)
    xn = jnp.sum(x * x, axis=1, keepdims=True)
    d = xn + cbn_scr[...] - 2.0 * xdotc
    minval = jnp.min(d, axis=1, keepdims=True)
    ids = lax.broadcasted_iota(jnp.int32, d.shape, 1)
    idx = jnp.min(jnp.where(d == minval, ids, _K), axis=1)
    idx_ref[...] = idx.reshape(_BLK // _IDXW, _IDXW)
    acc_scr[...] += jnp.sum(minval, axis=0, keepdims=True)

    @pl.when(i == _NBLK - 1)
    def _():
        loss_ref[...] = acc_scr[...] * (1.0 / (_B * _D))


_dist = pl.pallas_call(
    _dist_body,
    grid=(_NBLK,),
    in_specs=[
        pl.BlockSpec((_BLK, _D), lambda i: (i, 0)),
        pl.BlockSpec((_K, _D), lambda i: (0, 0)),
        pl.BlockSpec((_D, _H), lambda i: (0, 0)),
        pl.BlockSpec((1, _H), lambda i: (0, 0)),
    ],
    out_specs=[
        pl.BlockSpec((_BLK // _IDXW, _IDXW), lambda i: (i, 0)),
        pl.BlockSpec((1, 1), lambda i: (0, 0)),
        pl.BlockSpec((_K, _HP), lambda i: (0, 0)),
    ],
    out_shape=[
        jax.ShapeDtypeStruct((_B // _IDXW, _IDXW), jnp.int32),
        jax.ShapeDtypeStruct((1, 1), jnp.float32),
        jax.ShapeDtypeStruct((_K, _HP), jnp.float32),
    ],
    scratch_shapes=[pltpu.VMEM((1, _K), jnp.float32),
                    pltpu.VMEM((1, 1), jnp.float32)],
    compiler_params=pltpu.CompilerParams(skip_device_barrier=True),
)


@functools.cache
def _make_sc_gather():
    @functools.partial(
        pl.kernel,
        mesh=plsc.VectorSubcoreMesh(core_axis_name="c", subcore_axis_name="s",
                                    num_cores=_NC, num_subcores=_NS),
        out_type=jax.ShapeDtypeStruct((_B, _HP), jnp.float32),
        scratch_types=[
            pltpu.VMEM((_GC, _GS), jnp.int32),
            pltpu.VMEM((_BPW, _HP), jnp.float32),
            [pltpu.SemaphoreType.DMA] * _GC,
            [pltpu.SemaphoreType.DMA] * _GC,
            [pltpu.SemaphoreType.DMA] * _GC,
        ],
    )
    def _sc_gather(table_hbm, idx_hbm, out_hbm, idx_v, rows_v,
                   isems, gsems, wsems):
        wid = lax.axis_index("s") * _NC + lax.axis_index("c")
        base = wid * _BPW
        iloads = [pltpu.async_copy(
            idx_hbm.at[pl.ds(base + c * _GS, _GS)], idx_v.at[c], isems[c])
            for c in range(_GC)]
        gathers = []
        for c in range(_GC):
            iloads[c].wait()
            gathers.append(pltpu.async_copy(
                table_hbm.at[idx_v.at[c]],
                rows_v.at[pl.ds(c * _GS, _GS)], gsems[c]))
        wbs = []
        for c in range(_GC):
            gathers[c].wait()
            wbs.append(pltpu.async_copy(
                rows_v.at[pl.ds(c * _GS, _GS)],
                out_hbm.at[pl.ds(base + c * _GS, _GS)], wsems[c]))
        for c in range(_GC):
            wbs[c].wait()

    return _sc_gather


def _mlp_body(g_ref, W2_ref, b2_ref, W3_ref, b3_ref, out_ref):
    h1 = jnp.tanh(g_ref[:, :_H])
    h2 = jnp.tanh(jnp.dot(h1, W2_ref[...],
                          preferred_element_type=jnp.float32) + b2_ref[...])
    out_ref[...] = (jnp.dot(h2, W3_ref[...],
                            preferred_element_type=jnp.float32) + b3_ref[...])


_mlp = pl.pallas_call(
    _mlp_body,
    out_shape=jax.ShapeDtypeStruct((_B, _A), jnp.float32),
    compiler_params=pltpu.CompilerParams(skip_device_barrier=True),
)


@jax.jit
def kernel(x, codebook, W1, b1, W2, b2, W3, b3):
    idx2d, loss, cbw1b = _dist(x, codebook, W1, b1.reshape(1, _H))
    g = _make_sc_gather()(cbw1b, idx2d.reshape(_B))
    dist = _mlp(g, W2, b2.reshape(1, _H), W3, b3.reshape(1, _A))
    return dist, loss.reshape(())
